# Initial kernel scaffold; baseline (speedup 1.0000x reference)
import jax
import jax.numpy as jnp
from jax.experimental import pallas as pl


def kernel(adjacency, input_feature, weight):
    raise NotImplementedError("write your pallas kernel here")


# speedup vs baseline: 1.0016x; 1.0016x over previous
import jax
import jax.numpy as jnp
from jax.experimental import pallas as pl
from jax.experimental.pallas import tpu as pltpu


def _support_body(x_ref, w_ref, s_ref):
    s_ref[...] = jnp.dot(
        x_ref[...], w_ref[...], preferred_element_type=jnp.float32
    ).astype(jnp.bfloat16)


def _gemm_body(a_ref, s_ref, o_ref):
    a = a_ref[...].astype(jnp.bfloat16)
    o_ref[...] = jnp.dot(a, s_ref[...], preferred_element_type=jnp.float32)


def _pick_block(m: int, target: int = 256) -> int:
    best = 0
    for bm in range(8, min(m, target) + 1, 8):
        if m % bm == 0:
            best = bm
    return best


def kernel(adjacency, input_feature, weight):
    m, k = adjacency.shape
    k2, d_in = input_feature.shape
    d_out = weight.shape[1]
    assert k == k2

    support = pl.pallas_call(
        _support_body,
        out_shape=jax.ShapeDtypeStruct((k, d_out), jnp.bfloat16),
    )(input_feature, weight)

    bm = _pick_block(m)
    if bm == 0:
        bm = m
    grid = (m // bm,)

    out = pl.pallas_call(
        _gemm_body,
        grid=grid,
        in_specs=[
            pl.BlockSpec((bm, k), lambda i: (i, 0)),
            pl.BlockSpec((k, d_out), lambda i: (0, 0)),
        ],
        out_specs=pl.BlockSpec((bm, d_out), lambda i: (i, 0)),
        out_shape=jax.ShapeDtypeStruct((m, d_out), jnp.float32),
        compiler_params=pltpu.CompilerParams(
            dimension_semantics=("parallel",),
        ),
    )(adjacency, support)
    return out


# speedup vs baseline: 1.0284x; 1.0268x over previous
import jax
import jax.numpy as jnp
from jax.experimental import pallas as pl
from jax.experimental.pallas import tpu as pltpu


def _body(x_ref, w_ref, a_ref, o_ref, s_ref):
    @pl.when(pl.program_id(0) == 0)
    def _():
        s_ref[...] = jnp.dot(
            x_ref[...], w_ref[...], preferred_element_type=jnp.float32
        ).astype(jnp.bfloat16)

    a = a_ref[...].astype(jnp.bfloat16)
    o_ref[...] = jnp.dot(a, s_ref[...], preferred_element_type=jnp.float32)


def _pick_block(m: int, target: int = 256) -> int:
    best = 0
    for bm in range(8, min(m, target) + 1, 8):
        if m % bm == 0:
            best = bm
    return best


def kernel(adjacency, input_feature, weight):
    m, k = adjacency.shape
    k2, d_in = input_feature.shape
    d_out = weight.shape[1]
    assert k == k2

    bm = _pick_block(m)
    if bm == 0:
        bm = m

    out = pl.pallas_call(
        _body,
        grid=(m // bm,),
        in_specs=[
            pl.BlockSpec((k, d_in), lambda i: (0, 0)),
            pl.BlockSpec((d_in, d_out), lambda i: (0, 0)),
            pl.BlockSpec((bm, k), lambda i: (i, 0)),
        ],
        out_specs=pl.BlockSpec((bm, d_out), lambda i: (i, 0)),
        out_shape=jax.ShapeDtypeStruct((m, d_out), jnp.float32),
        scratch_shapes=[pltpu.VMEM((k, d_out), jnp.bfloat16)],
        compiler_params=pltpu.CompilerParams(
            dimension_semantics=("arbitrary",),
        ),
    )(input_feature, weight, adjacency)
    return out


# speedup vs baseline: 1.0358x; 1.0072x over previous
import jax
import jax.numpy as jnp
from jax.experimental import pallas as pl
from jax.experimental.pallas import tpu as pltpu


def _body(x_ref, w_ref, a_ref, o_ref, s_ref):
    @pl.when(pl.program_id(0) == 0)
    def _():
        s_ref[...] = jnp.dot(
            x_ref[...], w_ref[...], preferred_element_type=jnp.float32
        ).astype(jnp.bfloat16)

    a = a_ref[...].astype(jnp.bfloat16)
    o_ref[...] = jnp.dot(a, s_ref[...], preferred_element_type=jnp.float32)


def _pick_block(m: int, target: int = 400) -> int:
    best = 0
    for bm in range(8, min(m, target) + 1, 8):
        if m % bm == 0:
            best = bm
    return best


def kernel(adjacency, input_feature, weight):
    m, k = adjacency.shape
    k2, d_in = input_feature.shape
    d_out = weight.shape[1]
    assert k == k2

    bm = _pick_block(m)
    if bm == 0:
        bm = m

    out = pl.pallas_call(
        _body,
        grid=(m // bm,),
        in_specs=[
            pl.BlockSpec((k, d_in), lambda i: (0, 0)),
            pl.BlockSpec((d_in, d_out), lambda i: (0, 0)),
            pl.BlockSpec((bm, k), lambda i: (i, 0)),
        ],
        out_specs=pl.BlockSpec((bm, d_out), lambda i: (i, 0)),
        out_shape=jax.ShapeDtypeStruct((m, d_out), jnp.float32),
        scratch_shapes=[pltpu.VMEM((k, d_out), jnp.bfloat16)],
        compiler_params=pltpu.CompilerParams(
            dimension_semantics=("arbitrary",),
        ),
    )(input_feature, weight, adjacency)
    return out


# speedup vs baseline: 1.0416x; 1.0056x over previous
import jax
import jax.numpy as jnp
from jax.experimental import pallas as pl
from jax.experimental.pallas import tpu as pltpu


def _body(x_ref, w_ref, a_ref, o_ref, s_ref):
    @pl.when(pl.program_id(0) == 0)
    def _():
        s_ref[...] = jnp.dot(
            x_ref[...], w_ref[...], preferred_element_type=jnp.float32
        )

    o_ref[...] = jnp.dot(
        a_ref[...], s_ref[...], preferred_element_type=jnp.float32,
        precision=jax.lax.Precision.DEFAULT,
    )


def _pick_block(m: int, target: int = 400) -> int:
    best = 0
    for bm in range(8, min(m, target) + 1, 8):
        if m % bm == 0:
            best = bm
    return best


def kernel(adjacency, input_feature, weight):
    m, k = adjacency.shape
    k2, d_in = input_feature.shape
    d_out = weight.shape[1]
    assert k == k2

    bm = _pick_block(m)
    if bm == 0:
        bm = m

    out = pl.pallas_call(
        _body,
        grid=(m // bm,),
        in_specs=[
            pl.BlockSpec((k, d_in), lambda i: (0, 0)),
            pl.BlockSpec((d_in, d_out), lambda i: (0, 0)),
            pl.BlockSpec((bm, k), lambda i: (i, 0)),
        ],
        out_specs=pl.BlockSpec((bm, d_out), lambda i: (i, 0)),
        out_shape=jax.ShapeDtypeStruct((m, d_out), jnp.float32),
        scratch_shapes=[pltpu.VMEM((k, d_out), jnp.float32)],
        compiler_params=pltpu.CompilerParams(
            dimension_semantics=("arbitrary",),
        ),
    )(input_feature, weight, adjacency)
    return out
